# Initial kernel scaffold; baseline (speedup 1.0000x reference)
import jax
import jax.numpy as jnp
from jax.experimental import pallas as pl


def kernel(x, LUT):
    raise NotImplementedError("write your pallas kernel here")


# speedup vs baseline: 42.9171x; 42.9171x over previous
import functools

import jax
import jax.numpy as jnp
from jax import lax
from jax.experimental import pallas as pl
from jax.experimental.pallas import tpu as pltpu
from jax.experimental.pallas import tpu_sc as plsc

DIM = 17
TBL = DIM ** 4
CHUNK = 2048
SUB = 128
NSUB = CHUNK // SUB
NGRP = SUB // 16
PX_PER_TILE = 65536
NCHUNK = PX_PER_TILE // CHUNK


def _build_table(LUT):
    lutT = jnp.transpose(LUT[0], (1, 2, 3, 4, 0))
    Xp = jnp.pad(lutT, ((0, 1), (0, 1), (0, 1), (0, 1), (0, 0)))
    rows = jnp.stack(
        [Xp[(n & 1):(n & 1) + DIM,
            ((n >> 1) & 1):((n >> 1) & 1) + DIM,
            ((n >> 2) & 1):((n >> 2) & 1) + DIM,
            ((n >> 3) & 1):((n >> 3) & 1) + DIM] for n in range(16)],
        axis=4)
    return rows.reshape(TBL, 64)


def kernel(x, LUT):
    B, C, H, W = x.shape
    xr = x.reshape(B, C, H * W)
    table = _build_table(LUT)
    info = plsc.get_sparse_core_info()
    NC = info.num_cores

    mesh = plsc.VectorSubcoreMesh(core_axis_name="c", subcore_axis_name="s")

    @functools.partial(
        pl.kernel,
        mesh=mesh,
        out_type=jax.ShapeDtypeStruct((B, C, H * W), jnp.float32),
        scratch_types=[
            pltpu.VMEM((4, CHUNK), jnp.float32),
            pltpu.VMEM((4, CHUNK), jnp.float32),
            pltpu.VMEM((NSUB, SUB), jnp.int32),
            pltpu.VMEM((SUB, 64), jnp.float32),
            pltpu.SemaphoreType.DMA,
        ],
        compiler_params=pltpu.CompilerParams(
            needs_layout_passes=False, use_tc_tiling_on_sc=False),
    )
    def sc_kernel(x_hbm, tbl_hbm, out_hbm, xbuf, outbuf, idxbuf, rowbuf, sem):
        wid = lax.axis_index("s") * NC + lax.axis_index("c")
        b = wid // 4
        base = (wid % 4) * PX_PER_TILE
        iota = lax.iota(jnp.int32, 16)

        def frac_parts(q):
            d = []
            for c in range(4):
                xv = xbuf[c, pl.ds(q, 16)]
                xv = jnp.minimum(jnp.maximum(xv, 0.0), 1.0)
                posv = xv * jnp.float32(DIM - 1)
                fi = jnp.minimum(posv.astype(jnp.int32), DIM - 2)
                d.append((fi, posv - fi.astype(jnp.float32)))
            return d

        def chunk_body(g, _):
            start = base + g * CHUNK
            for c in range(4):
                pltpu.sync_copy(x_hbm.at[b, c, pl.ds(start, CHUNK)],
                                xbuf.at[c])

            def idx_body(j, _):
                def grp_body(k, _):
                    q = j * SUB + k * 16
                    parts = frac_parts(q)
                    idx = parts[0][0]
                    for c in range(1, 4):
                        idx = idx * DIM + parts[c][0]
                    idxbuf[j, pl.ds(k * 16, 16)] = idx
                    return _
                return lax.fori_loop(0, NGRP, grp_body, _)
            lax.fori_loop(0, NSUB, idx_body, None)

            def sub_body(j, _):
                pltpu.async_copy(tbl_hbm.at[idxbuf.at[j]], rowbuf, sem).wait()

                def grp_body(k, _):
                    q = j * SUB + k * 16
                    d = [p[1] for p in frac_parts(q)]
                    wa, wc = [], []
                    for m in range(4):
                        f0 = d[0] if (m & 1) else 1.0 - d[0]
                        f1 = d[1] if (m & 2) else 1.0 - d[1]
                        wa.append(f0 * f1)
                        f2 = d[2] if (m & 1) else 1.0 - d[2]
                        f3 = d[3] if (m & 2) else 1.0 - d[3]
                        wc.append(f2 * f3)
                    rowv = k * 16 + iota
                    acc = [jnp.zeros((16,), jnp.float32) for _ in range(4)]
                    for n in range(16):
                        w = wa[n & 3] * wc[(n >> 2) & 3]
                        for c in range(4):
                            col = jnp.full((16,), n * 4 + c, jnp.int32)
                            gv = plsc.load_gather(rowbuf, [rowv, col])
                            acc[c] = acc[c] + w * gv
                    for c in range(4):
                        outbuf[c, pl.ds(q, 16)] = acc[c]
                    return _
                return lax.fori_loop(0, NGRP, grp_body, _)
            lax.fori_loop(0, NSUB, sub_body, None)

            for c in range(4):
                pltpu.sync_copy(outbuf.at[c],
                                out_hbm.at[b, c, pl.ds(start, CHUNK)])
            return _
        lax.fori_loop(0, NCHUNK, chunk_body, None)

    out = sc_kernel(xr, table)
    return out.reshape(B, C, H, W)


# speedup vs baseline: 51.4724x; 1.1993x over previous
import functools

import jax
import jax.numpy as jnp
from jax import lax
from jax.experimental import pallas as pl
from jax.experimental.pallas import tpu as pltpu
from jax.experimental.pallas import tpu_sc as plsc

DIM = 17
TBL = DIM ** 4
CHUNK = 2048
SUB = 128
NSUB = CHUNK // SUB
NGRP = SUB // 16
PX_PER_TILE = 65536
NCHUNK = PX_PER_TILE // CHUNK


def _build_table(LUT):
    lutT = jnp.transpose(LUT[0], (1, 2, 3, 4, 0))
    Xp = jnp.pad(lutT, ((0, 1), (0, 1), (0, 1), (0, 1), (0, 0)))
    rows = jnp.stack(
        [Xp[(n & 1):(n & 1) + DIM,
            ((n >> 1) & 1):((n >> 1) & 1) + DIM,
            ((n >> 2) & 1):((n >> 2) & 1) + DIM,
            ((n >> 3) & 1):((n >> 3) & 1) + DIM] for n in range(16)],
        axis=4)
    return rows.reshape(TBL, 64)


def kernel(x, LUT):
    B, C, H, W = x.shape
    xr = x.reshape(B, C, H * W)
    table = _build_table(LUT)
    info = plsc.get_sparse_core_info()
    NC = info.num_cores

    mesh = plsc.VectorSubcoreMesh(core_axis_name="c", subcore_axis_name="s")

    @functools.partial(
        pl.kernel,
        mesh=mesh,
        out_type=jax.ShapeDtypeStruct((B, C, H * W), jnp.float32),
        scratch_types=[
            pltpu.VMEM((4, CHUNK), jnp.float32),
            pltpu.VMEM((4, CHUNK), jnp.float32),
            pltpu.VMEM((NSUB, SUB), jnp.int32),
            pltpu.VMEM((2, SUB, 64), jnp.float32),
            pltpu.SemaphoreType.DMA,
            pltpu.SemaphoreType.DMA,
            pltpu.SemaphoreType.DMA,
        ],
        compiler_params=pltpu.CompilerParams(
            needs_layout_passes=False, use_tc_tiling_on_sc=False),
    )
    def sc_kernel(x_hbm, tbl_hbm, out_hbm, xbuf, outbuf, idxbuf, rowbuf,
                  semx, semg0, semg1):
        wid = lax.axis_index("s") * NC + lax.axis_index("c")
        b = wid // 4
        base = (wid % 4) * PX_PER_TILE
        iota = lax.iota(jnp.int32, 16)

        def frac_parts(q):
            d = []
            for c in range(4):
                xv = xbuf[c, pl.ds(q, 16)]
                xv = jnp.minimum(jnp.maximum(xv, 0.0), 1.0)
                posv = xv * jnp.float32(DIM - 1)
                fi = jnp.minimum(posv.astype(jnp.int32), DIM - 2)
                d.append((fi, posv - fi.astype(jnp.float32)))
            return d

        def fire(j, p, sem):
            pltpu.async_copy(tbl_hbm.at[idxbuf.at[j]], rowbuf.at[p], sem)

        def drain(p, sem):
            pltpu.make_async_copy(tbl_hbm.at[idxbuf.at[0]], rowbuf.at[p],
                                  sem).wait()

        def compute_sub(j, p):
            def grp_body(k, _):
                q = j * SUB + k * 16
                d = [pp[1] for pp in frac_parts(q)]
                wa, wc = [], []
                for m in range(4):
                    f0 = d[0] if (m & 1) else 1.0 - d[0]
                    f1 = d[1] if (m & 2) else 1.0 - d[1]
                    wa.append(f0 * f1)
                    f2 = d[2] if (m & 1) else 1.0 - d[2]
                    f3 = d[3] if (m & 2) else 1.0 - d[3]
                    wc.append(f2 * f3)
                rowv = k * 16 + iota
                acc = [jnp.zeros((16,), jnp.float32) for _ in range(4)]
                for n in range(16):
                    w = wa[n & 3] * wc[(n >> 2) & 3]
                    for c in range(4):
                        col = jnp.full((16,), n * 4 + c, jnp.int32)
                        gv = plsc.load_gather(rowbuf.at[p], [rowv, col])
                        acc[c] = acc[c] + w * gv
                for c in range(4):
                    outbuf[c, pl.ds(q, 16)] = acc[c]
                return _
            lax.fori_loop(0, NGRP, grp_body, None)

        def chunk_body(g, _):
            start = base + g * CHUNK
            xd = [pltpu.async_copy(x_hbm.at[b, c, pl.ds(start, CHUNK)],
                                   xbuf.at[c], semx) for c in range(4)]
            for dsc in xd:
                dsc.wait()

            def idx_body(j, _):
                def grp_body(k, _):
                    q = j * SUB + k * 16
                    parts = frac_parts(q)
                    idx = parts[0][0]
                    for c in range(1, 4):
                        idx = idx * DIM + parts[c][0]
                    idxbuf[j, pl.ds(k * 16, 16)] = idx
                    return _
                return lax.fori_loop(0, NGRP, grp_body, _)
            lax.fori_loop(0, NSUB, idx_body, None)

            fire(0, 0, semg0)

            def pair_body(t, _):
                j0 = 2 * t
                fire(j0 + 1, 1, semg1)
                drain(0, semg0)
                compute_sub(j0, 0)
                fire(j0 + 2, 0, semg0)
                drain(1, semg1)
                compute_sub(j0 + 1, 1)
                return _
            lax.fori_loop(0, NSUB // 2 - 1, pair_body, None)
            fire(NSUB - 1, 1, semg1)
            drain(0, semg0)
            compute_sub(NSUB - 2, 0)
            drain(1, semg1)
            compute_sub(NSUB - 1, 1)

            for c in range(4):
                pltpu.sync_copy(outbuf.at[c],
                                out_hbm.at[b, c, pl.ds(start, CHUNK)])
            return _
        lax.fori_loop(0, NCHUNK, chunk_body, None)

    out = sc_kernel(xr, table)
    return out.reshape(B, C, H, W)


# speedup vs baseline: 91.9137x; 1.7857x over previous
import functools

import jax
import jax.numpy as jnp
from jax import lax
from jax.experimental import pallas as pl
from jax.experimental.pallas import tpu as pltpu
from jax.experimental.pallas import tpu_sc as plsc

DIM = 17
TBL = DIM ** 4
CHUNK = 2048
SUB = 128
NSUB = CHUNK // SUB
NGRP = SUB // 16
PX_PER_TILE = 65536
NCHUNK = PX_PER_TILE // CHUNK


def _build_table(LUT):
    lutT = jnp.transpose(LUT[0], (1, 2, 3, 4, 0))
    Xp = jnp.pad(lutT, ((0, 1), (0, 1), (0, 1), (0, 1), (0, 0)))
    rows = jnp.stack(
        [Xp[(n & 1):(n & 1) + DIM,
            ((n >> 1) & 1):((n >> 1) & 1) + DIM,
            ((n >> 2) & 1):((n >> 2) & 1) + DIM,
            ((n >> 3) & 1):((n >> 3) & 1) + DIM] for n in range(16)],
        axis=4)
    return rows.reshape(TBL, 64)


def kernel(x, LUT):
    B, C, H, W = x.shape
    xr = x.reshape(B, C, H * W)
    table = _build_table(LUT)
    info = plsc.get_sparse_core_info()
    NC = info.num_cores

    mesh = plsc.VectorSubcoreMesh(core_axis_name="c", subcore_axis_name="s")

    @functools.partial(
        pl.kernel,
        mesh=mesh,
        out_type=jax.ShapeDtypeStruct((B, C, H * W), jnp.float32),
        scratch_types=[
            pltpu.VMEM((4, CHUNK), jnp.float32),
            pltpu.VMEM((4, CHUNK), jnp.float32),
            pltpu.VMEM((4, CHUNK), jnp.float32),
            pltpu.VMEM((NSUB, SUB), jnp.int32),
            pltpu.VMEM((2, SUB, 64), jnp.float32),
            pltpu.VMEM((CHUNK * 4,), jnp.float32),
            pltpu.SemaphoreType.DMA,
            pltpu.SemaphoreType.DMA,
            pltpu.SemaphoreType.DMA,
        ],
        compiler_params=pltpu.CompilerParams(
            needs_layout_passes=False, use_tc_tiling_on_sc=False),
    )
    def sc_kernel(x_hbm, tbl_hbm, out_hbm, xbuf, outbuf, dbuf, idxbuf,
                  rowbuf, aosbuf, semx, semg0, semg1):
        wid = lax.axis_index("s") * NC + lax.axis_index("c")
        b = wid // 4
        base = (wid % 4) * PX_PER_TILE
        iota = lax.iota(jnp.int32, 16)
        rot4 = (iota + 4) & 15
        rot8 = (iota + 8) & 15
        rot12 = (iota + 12) & 15
        m4 = iota < 4
        m8 = iota < 8
        m12 = iota < 12

        def fire(j, p, sem):
            pltpu.async_copy(tbl_hbm.at[idxbuf.at[j]], rowbuf.at[p], sem)

        def drain(p, sem):
            pltpu.make_async_copy(tbl_hbm.at[idxbuf.at[0]], rowbuf.at[p],
                                  sem).wait()

        def compute_sub(j, p):
            def grp_body(k, _):
                qv = j * SUB + k * 16
                dv = [dbuf[c, pl.ds(qv, 16)] for c in range(4)]
                merged = []
                for pp in range(4):
                    quart = []
                    for p4 in range(4):
                        pix = pp * 4 + p4
                        r = k * 16 + pix
                        g0 = rowbuf[p, r, pl.ds(0, 16)]
                        g1 = rowbuf[p, r, pl.ds(16, 16)]
                        g2 = rowbuf[p, r, pl.ds(32, 16)]
                        g3 = rowbuf[p, r, pl.ds(48, 16)]
                        d0 = dv[0][pix]
                        d1 = dv[1][pix]
                        d2 = dv[2][pix]
                        d3 = dv[3][pix]
                        u0 = g0 * (1.0 - d3) + g2 * d3
                        u1 = g1 * (1.0 - d3) + g3 * d3
                        s = u0 * (1.0 - d2) + u1 * d2
                        s = (s * (1.0 - d1) +
                             jnp.take(s, rot8, mode="wrap") * d1)
                        s = (s * (1.0 - d0) +
                             jnp.take(s, rot4, mode="wrap") * d0)
                        quart.append(s)
                    m = jnp.where(
                        m4, quart[0],
                        jnp.take(quart[1], rot12, mode="wrap"))
                    m = jnp.where(
                        m8, m,
                        jnp.take(quart[2], rot8, mode="wrap"))
                    m = jnp.where(
                        m12, m,
                        jnp.take(quart[3], rot4, mode="wrap"))
                    merged.append((pp, m))
                qbase = (j * SUB + k * 16) * 4
                for pp, m in merged:
                    aosbuf[pl.ds(qbase + pp * 16, 16)] = m
                return _
            lax.fori_loop(0, NGRP, grp_body, None)

        def chunk_body(g, _):
            start = base + g * CHUNK
            xd = [pltpu.async_copy(x_hbm.at[b, c, pl.ds(start, CHUNK)],
                                   xbuf.at[c], semx) for c in range(4)]
            for dsc in xd:
                dsc.wait()

            def idx_body(j, _):
                def grp_body(k, _):
                    q = j * SUB + k * 16
                    idx = None
                    for c in range(4):
                        xv = xbuf[c, pl.ds(q, 16)]
                        xv = jnp.minimum(jnp.maximum(xv, 0.0), 1.0)
                        posv = xv * jnp.float32(DIM - 1)
                        fi = jnp.minimum(posv.astype(jnp.int32), DIM - 2)
                        dbuf[c, pl.ds(q, 16)] = posv - fi.astype(jnp.float32)
                        idx = fi if idx is None else idx * DIM + fi
                    idxbuf[j, pl.ds(k * 16, 16)] = idx
                    return _
                return lax.fori_loop(0, NGRP, grp_body, _)
            lax.fori_loop(0, NSUB, idx_body, None)

            fire(0, 0, semg0)

            def pair_body(t, _):
                j0 = 2 * t
                fire(j0 + 1, 1, semg1)
                drain(0, semg0)
                compute_sub(j0, 0)
                fire(j0 + 2, 0, semg0)
                drain(1, semg1)
                compute_sub(j0 + 1, 1)
                return _
            lax.fori_loop(0, NSUB // 2 - 1, pair_body, None)
            fire(NSUB - 1, 1, semg1)
            drain(0, semg0)
            compute_sub(NSUB - 2, 0)
            drain(1, semg1)
            compute_sub(NSUB - 1, 1)

            def tr_body(k, _):
                addr = (k * 16 + iota) * 4
                for c in range(4):
                    outbuf[c, pl.ds(k * 16, 16)] = plsc.load_gather(
                        aosbuf, [addr + c])
                return _
            lax.fori_loop(0, CHUNK // 16, tr_body, None)

            for c in range(4):
                pltpu.sync_copy(outbuf.at[c],
                                out_hbm.at[b, c, pl.ds(start, CHUNK)])
            return _
        lax.fori_loop(0, NCHUNK, chunk_body, None)

    out = sc_kernel(xr, table)
    return out.reshape(B, C, H, W)


# speedup vs baseline: 114.2276x; 1.2428x over previous
import functools

import jax
import jax.numpy as jnp
from jax import lax
from jax.experimental import pallas as pl
from jax.experimental.pallas import tpu as pltpu
from jax.experimental.pallas import tpu_sc as plsc

DIM = 17
TBL = DIM ** 4
CHUNK = 2048
SUB = 128
NSUB = CHUNK // SUB
NGRP = SUB // 16
PX_PER_TILE = 65536
NCHUNK = PX_PER_TILE // CHUNK
OFF = (0, DIM ** 2, DIM ** 3, DIM ** 3 + DIM ** 2)


def _build_table(LUT):
    lutT = jnp.transpose(LUT[0], (1, 2, 3, 4, 0))
    Xp = jnp.pad(lutT, ((0, 0), (0, 0), (0, 1), (0, 1), (0, 0)))
    rows = jnp.stack(
        [Xp[:, :, (n & 1):(n & 1) + DIM, (n >> 1):(n >> 1) + DIM]
         for n in range(4)],
        axis=4)
    return rows.reshape(TBL, 16)


def kernel(x, LUT):
    B, C, H, W = x.shape
    xr = x.reshape(B, C, H * W)
    table = _build_table(LUT)
    info = plsc.get_sparse_core_info()
    NC = info.num_cores

    mesh = plsc.VectorSubcoreMesh(core_axis_name="c", subcore_axis_name="s")

    @functools.partial(
        pl.kernel,
        mesh=mesh,
        out_type=jax.ShapeDtypeStruct((B, C, H * W), jnp.float32),
        scratch_types=[
            pltpu.VMEM((4, CHUNK), jnp.float32),
            pltpu.VMEM((4, CHUNK), jnp.float32),
            pltpu.VMEM((4, CHUNK), jnp.float32),
            pltpu.VMEM((NSUB, 4, SUB), jnp.int32),
            pltpu.VMEM((2, SUB * 4, 16), jnp.float32),
            pltpu.VMEM((CHUNK * 4,), jnp.float32),
            pltpu.SemaphoreType.DMA,
            pltpu.SemaphoreType.DMA,
            pltpu.SemaphoreType.DMA,
        ],
        compiler_params=pltpu.CompilerParams(
            needs_layout_passes=False, use_tc_tiling_on_sc=False),
    )
    def sc_kernel(x_hbm, tbl_hbm, out_hbm, xbuf, outbuf, dbuf, idxbuf,
                  rowbuf, aosbuf, semx, semg0, semg1):
        wid = lax.axis_index("s") * NC + lax.axis_index("c")
        b = wid // 4
        base = (wid % 4) * PX_PER_TILE
        iota = lax.iota(jnp.int32, 16)
        rot4 = (iota + 4) & 15
        rot8 = (iota + 8) & 15
        rot12 = (iota + 12) & 15
        m4 = iota < 4
        m8 = iota < 8
        m12 = iota < 12

        def fire(j, p, sem):
            for m in range(4):
                pltpu.async_copy(tbl_hbm.at[idxbuf.at[j, m]],
                                 rowbuf.at[p, pl.ds(m * SUB, SUB)], sem)

        def drain(p, sem):
            pltpu.make_async_copy(tbl_hbm.at[idxbuf.at[0, 0]], rowbuf.at[p],
                                  sem).wait()

        def compute_sub(j, p):
            def grp_body(k, _):
                qv = j * SUB + k * 16
                dv = [dbuf[c, pl.ds(qv, 16)] for c in range(4)]
                merged = []
                for pp in range(4):
                    quart = []
                    for p4 in range(4):
                        pix = pp * 4 + p4
                        r = k * 16 + pix
                        v0 = rowbuf[p, r, :]
                        v1 = rowbuf[p, SUB + r, :]
                        v2 = rowbuf[p, 2 * SUB + r, :]
                        v3 = rowbuf[p, 3 * SUB + r, :]
                        d0 = dv[0][pix]
                        d1 = dv[1][pix]
                        d2 = dv[2][pix]
                        d3 = dv[3][pix]
                        a0 = v0 * (1.0 - d1) + v1 * d1
                        a1 = v2 * (1.0 - d1) + v3 * d1
                        s = a0 * (1.0 - d0) + a1 * d0
                        s = (s * (1.0 - d3) +
                             jnp.take(s, rot8, mode="wrap") * d3)
                        s = (s * (1.0 - d2) +
                             jnp.take(s, rot4, mode="wrap") * d2)
                        quart.append(s)
                    m = jnp.where(
                        m4, quart[0],
                        jnp.take(quart[1], rot12, mode="wrap"))
                    m = jnp.where(
                        m8, m,
                        jnp.take(quart[2], rot8, mode="wrap"))
                    m = jnp.where(
                        m12, m,
                        jnp.take(quart[3], rot4, mode="wrap"))
                    merged.append((pp, m))
                qbase = (j * SUB + k * 16) * 4
                for pp, m in merged:
                    aosbuf[pl.ds(qbase + pp * 16, 16)] = m
                return _
            lax.fori_loop(0, NGRP, grp_body, None)

        def chunk_body(g, _):
            start = base + g * CHUNK
            xd = [pltpu.async_copy(x_hbm.at[b, c, pl.ds(start, CHUNK)],
                                   xbuf.at[c], semx) for c in range(4)]
            for dsc in xd:
                dsc.wait()

            def idx_body(j, _):
                def grp_body(k, _):
                    q = j * SUB + k * 16
                    idx = None
                    for c in range(4):
                        xv = xbuf[c, pl.ds(q, 16)]
                        xv = jnp.minimum(jnp.maximum(xv, 0.0), 1.0)
                        posv = xv * jnp.float32(DIM - 1)
                        fi = jnp.minimum(posv.astype(jnp.int32), DIM - 2)
                        dbuf[c, pl.ds(q, 16)] = posv - fi.astype(jnp.float32)
                        idx = fi if idx is None else idx * DIM + fi
                    for m in range(4):
                        idxbuf[j, m, pl.ds(k * 16, 16)] = idx + OFF[m]
                    return _
                return lax.fori_loop(0, NGRP, grp_body, _)
            lax.fori_loop(0, NSUB, idx_body, None)

            fire(0, 0, semg0)

            def pair_body(t, _):
                j0 = 2 * t
                fire(j0 + 1, 1, semg1)
                drain(0, semg0)
                compute_sub(j0, 0)
                fire(j0 + 2, 0, semg0)
                drain(1, semg1)
                compute_sub(j0 + 1, 1)
                return _
            lax.fori_loop(0, NSUB // 2 - 1, pair_body, None)
            fire(NSUB - 1, 1, semg1)
            drain(0, semg0)
            compute_sub(NSUB - 2, 0)
            drain(1, semg1)
            compute_sub(NSUB - 1, 1)

            def tr_body(k, _):
                addr = (k * 16 + iota) * 4
                for c in range(4):
                    outbuf[c, pl.ds(k * 16, 16)] = plsc.load_gather(
                        aosbuf, [addr + c])
                return _
            lax.fori_loop(0, CHUNK // 16, tr_body, None)

            for c in range(4):
                pltpu.sync_copy(outbuf.at[c],
                                out_hbm.at[b, c, pl.ds(start, CHUNK)])
            return _
        lax.fori_loop(0, NCHUNK, chunk_body, None)

    out = sc_kernel(xr, table)
    return out.reshape(B, C, H, W)
